# baseline (speedup 1.0000x reference)
import functools
import jax
import jax.numpy as jnp
from jax import lax
from jax.experimental import pallas as pl
from jax.experimental.pallas import tpu as pltpu
from jax.experimental.pallas import tpu_sc as plsc

_BATCH = 4096
_ATOMS = 50
_DOUT = 640
_NROWS = 10

_NC = 2
_NS = 16
_NW = _NC * _NS

_B_PER_W = _BATCH // _NW
_LANES = 16
_VPR = _DOUT // _LANES


def _gate_body(idx_hbm, table_hbm, out_hbm, idx_v, table_v, buf0, buf1,
               wsem0, wsem1):
    wid = lax.axis_index("s") * _NC + lax.axis_index("c")
    base = wid * _B_PER_W

    pltpu.sync_copy(table_hbm, table_v)
    pltpu.sync_copy(idx_hbm.at[pl.ds(base * _ATOMS, _B_PER_W * _ATOMS)],
                    idx_v.at[pl.ds(0, _B_PER_W * _ATOMS)])

    bufs = (buf0, buf1)
    sems = (wsem0, wsem1)

    def build(c, buf):
        G = 8

        def row(r, carry):
            iv = idx_v[pl.ds(c * _ATOMS + r, _LANES)][0]
            vals = [table_v[iv, pl.ds(j * _LANES, _LANES)] for j in range(G)]
            for j in range(_VPR):
                buf[r, pl.ds(j * _LANES, _LANES)] = vals[j % G]
                if j + G < _VPR:
                    vals[j % G] = table_v[iv, pl.ds((j + G) * _LANES, _LANES)]
            return carry
        lax.fori_loop(0, _ATOMS, row, 0)

    def write(c, buf, sem):
        return pltpu.make_async_copy(buf, out_hbm.at[base + c], sem)

    for b in range(2):
        build(b, bufs[b])
        write(b, bufs[b], sems[b]).start()

    def step(i, carry):
        for b in range(2):
            c = 2 + 2 * i + b
            write(c - 2, bufs[b], sems[b]).wait()
            build(c, bufs[b])
            write(c, bufs[b], sems[b]).start()
        return carry

    lax.fori_loop(0, (_B_PER_W - 2) // 2, step, 0)

    for b in range(2):
        write(0, bufs[b], sems[b]).wait()


@jax.jit
def _gate_lookup(inputs, gate_weight):
    mesh = plsc.VectorSubcoreMesh(core_axis_name="c", subcore_axis_name="s")
    run = pl.kernel(
        _gate_body,
        out_type=jax.ShapeDtypeStruct((_BATCH, _ATOMS, _DOUT), jnp.float32),
        mesh=mesh,
        compiler_params=pltpu.CompilerParams(use_tc_tiling_on_sc=False),
        scratch_types=[
            pltpu.VMEM((_B_PER_W * _ATOMS + _LANES,), jnp.int32),
            pltpu.VMEM((_NROWS, _DOUT), jnp.float32),
            pltpu.VMEM((_ATOMS, _DOUT), jnp.float32),
            pltpu.VMEM((_ATOMS, _DOUT), jnp.float32),
            pltpu.SemaphoreType.DMA,
            pltpu.SemaphoreType.DMA,
        ],
    )
    return run(inputs.reshape(-1), gate_weight)


def kernel(inputs, gate_weight):
    return _gate_lookup(inputs, gate_weight)


# speedup vs baseline: 1.8907x; 1.8907x over previous
import functools
import jax
import jax.numpy as jnp
from jax import lax
from jax.experimental import pallas as pl
from jax.experimental.pallas import tpu as pltpu
from jax.experimental.pallas import tpu_sc as plsc

_BATCH = 4096
_ATOMS = 50
_DOUT = 640
_NROWS = 10

_NC = 2
_NS = 16
_NW = _NC * _NS

_B_PER_W = _BATCH // _NW
_LANES = 16
_VPR = _DOUT // _LANES


def _gate_body(idx_hbm, table_hbm, out_hbm, idx_v, table_v, buf0, buf1, buf2,
               wsem0, wsem1, wsem2):
    wid = lax.axis_index("s") * _NC + lax.axis_index("c")
    base = wid * _B_PER_W

    pltpu.sync_copy(table_hbm, table_v)
    pltpu.sync_copy(idx_hbm.at[pl.ds(base * _ATOMS, _B_PER_W * _ATOMS)],
                    idx_v.at[pl.ds(0, _B_PER_W * _ATOMS)])

    bufs = (buf0, buf1, buf2)
    sems = (wsem0, wsem1, wsem2)

    def build(c, buf):
        G = 8

        def row(r, carry):
            iv = idx_v[pl.ds(c * _ATOMS + r, _LANES)][0]
            vals = [table_v[iv, pl.ds(j * _LANES, _LANES)] for j in range(G)]
            for j in range(_VPR):
                buf[r, pl.ds(j * _LANES, _LANES)] = vals[j % G]
                if j + G < _VPR:
                    vals[j % G] = table_v[iv, pl.ds((j + G) * _LANES, _LANES)]
            return carry
        lax.fori_loop(0, _ATOMS, row, 0)

    def write(c, buf, sem):
        return pltpu.make_async_copy(buf, out_hbm.at[base + c], sem)

    for b in range(3):
        build(b, bufs[b])
        write(b, bufs[b], sems[b]).start()

    def step(i, carry):
        for b in range(3):
            c = 3 + 3 * i + b
            write(c - 3, bufs[b], sems[b]).wait()
            build(c, bufs[b])
            write(c, bufs[b], sems[b]).start()
        return carry

    lax.fori_loop(0, (_B_PER_W - 3) // 3, step, 0)
    for b in range(2):
        c = _B_PER_W - 2 + b
        write(c - 3, bufs[b], sems[b]).wait()
        build(c, bufs[b])
        write(c, bufs[b], sems[b]).start()

    for b in range(3):
        write(0, bufs[b], sems[b]).wait()


@jax.jit
def _gate_lookup(inputs, gate_weight):
    mesh = plsc.VectorSubcoreMesh(core_axis_name="c", subcore_axis_name="s")
    run = pl.kernel(
        _gate_body,
        out_type=jax.ShapeDtypeStruct((_BATCH, _ATOMS, _DOUT), jnp.float32),
        mesh=mesh,
        scratch_types=[
            pltpu.VMEM((_B_PER_W * _ATOMS + _LANES,), jnp.int32),
            pltpu.VMEM((_NROWS, _DOUT), jnp.float32),
            pltpu.VMEM((_ATOMS, _DOUT), jnp.float32),
            pltpu.VMEM((_ATOMS, _DOUT), jnp.float32),
            pltpu.VMEM((_ATOMS, _DOUT), jnp.float32),
            pltpu.SemaphoreType.DMA,
            pltpu.SemaphoreType.DMA,
            pltpu.SemaphoreType.DMA,
        ],
    )
    return run(inputs.reshape(-1), gate_weight)


def kernel(inputs, gate_weight):
    return _gate_lookup(inputs, gate_weight)


# speedup vs baseline: 2.1878x; 1.1571x over previous
import functools
import jax
import jax.numpy as jnp
from jax import lax
from jax.experimental import pallas as pl
from jax.experimental.pallas import tpu as pltpu
from jax.experimental.pallas import tpu_sc as plsc

_BATCH = 4096
_ATOMS = 50
_DOUT = 640
_NROWS = 10

_NC = 2
_NS = 16
_NW = _NC * _NS

_B_PER_W = _BATCH // _NW
_LANES = 16
_VPR = _DOUT // _LANES


def _gate_body(idx_hbm, table_hbm, out_hbm, idx_v, table_v, buf0, buf1, buf2,
               wsem0, wsem1, wsem2):
    wid = lax.axis_index("s") * _NC + lax.axis_index("c")
    base = wid * _B_PER_W

    pltpu.sync_copy(table_hbm, table_v)
    pltpu.sync_copy(idx_hbm.at[pl.ds(base * _ATOMS, _B_PER_W * _ATOMS)],
                    idx_v.at[pl.ds(0, _B_PER_W * _ATOMS)])

    bufs = (buf0, buf1, buf2)
    sems = (wsem0, wsem1, wsem2)

    def build(c, buf):
        G = 8
        R = 5

        def rows(g, carry):
            r0 = g * R
            ivs = [idx_v[pl.ds(c * _ATOMS + r0 + k, _LANES)][0]
                   for k in range(R)]
            slots = [(k, j) for k in range(R) for j in range(_VPR)]

            def load(p):
                k, j = slots[p]
                return table_v[ivs[k], pl.ds(j * _LANES, _LANES)]

            vals = [load(p) for p in range(G)]
            for p in range(R * _VPR):
                k, j = slots[p]
                buf[r0 + k, pl.ds(j * _LANES, _LANES)] = vals[p % G]
                if p + G < R * _VPR:
                    vals[p % G] = load(p + G)
            return carry
        lax.fori_loop(0, _ATOMS // R, rows, 0)

    def write(c, buf, sem):
        return pltpu.make_async_copy(buf, out_hbm.at[base + c], sem)

    for b in range(3):
        build(b, bufs[b])
        write(b, bufs[b], sems[b]).start()

    def step(i, carry):
        for b in range(3):
            c = 3 + 3 * i + b
            write(c - 3, bufs[b], sems[b]).wait()
            build(c, bufs[b])
            write(c, bufs[b], sems[b]).start()
        return carry

    lax.fori_loop(0, (_B_PER_W - 3) // 3, step, 0)
    for b in range(2):
        c = _B_PER_W - 2 + b
        write(c - 3, bufs[b], sems[b]).wait()
        build(c, bufs[b])
        write(c, bufs[b], sems[b]).start()

    for b in range(3):
        write(0, bufs[b], sems[b]).wait()


@jax.jit
def _gate_lookup(inputs, gate_weight):
    mesh = plsc.VectorSubcoreMesh(core_axis_name="c", subcore_axis_name="s")
    run = pl.kernel(
        _gate_body,
        out_type=jax.ShapeDtypeStruct((_BATCH, _ATOMS, _DOUT), jnp.float32),
        mesh=mesh,
        scratch_types=[
            pltpu.VMEM((_B_PER_W * _ATOMS + _LANES,), jnp.int32),
            pltpu.VMEM((_NROWS, _DOUT), jnp.float32),
            pltpu.VMEM((_ATOMS, _DOUT), jnp.float32),
            pltpu.VMEM((_ATOMS, _DOUT), jnp.float32),
            pltpu.VMEM((_ATOMS, _DOUT), jnp.float32),
            pltpu.SemaphoreType.DMA,
            pltpu.SemaphoreType.DMA,
            pltpu.SemaphoreType.DMA,
        ],
    )
    return run(inputs.reshape(-1), gate_weight)


def kernel(inputs, gate_weight):
    return _gate_lookup(inputs, gate_weight)


# speedup vs baseline: 2.2085x; 1.0094x over previous
import functools
import jax
import jax.numpy as jnp
from jax import lax
from jax.experimental import pallas as pl
from jax.experimental.pallas import tpu as pltpu
from jax.experimental.pallas import tpu_sc as plsc

_BATCH = 4096
_ATOMS = 50
_DOUT = 640
_NROWS = 10

_NC = 2
_NS = 16
_NW = _NC * _NS

_B_PER_W = _BATCH // _NW
_LANES = 16
_VPR = _DOUT // _LANES


def _gate_body(idx_hbm, table_hbm, out_hbm, idx_v, table_v, buf0, buf1, buf2,
               wsem0, wsem1, wsem2):
    wid = lax.axis_index("s") * _NC + lax.axis_index("c")

    pltpu.sync_copy(table_hbm, table_v)
    pltpu.sync_copy(idx_hbm.at[pl.ds(wid * _B_PER_W * _ATOMS,
                                     _B_PER_W * _ATOMS)],
                    idx_v.at[pl.ds(0, _B_PER_W * _ATOMS)])

    bufs = (buf0, buf1, buf2)
    sems = (wsem0, wsem1, wsem2)

    def build(c, buf):
        G = 8
        R = 5

        def rows(g, carry):
            r0 = g * R
            ivs = [idx_v[pl.ds(c * _ATOMS + r0 + k, _LANES)][0]
                   for k in range(R)]
            slots = [(k, j) for k in range(R) for j in range(_VPR)]

            def load(p):
                k, j = slots[p]
                return table_v[ivs[k], pl.ds(j * _LANES, _LANES)]

            vals = [load(p) for p in range(G)]
            for p in range(R * _VPR):
                k, j = slots[p]
                buf[r0 + k, pl.ds(j * _LANES, _LANES)] = vals[p % G]
                if p + G < R * _VPR:
                    vals[p % G] = load(p + G)
            return carry
        lax.fori_loop(0, _ATOMS // R, rows, 0)

    def write(c, buf, sem):
        return pltpu.make_async_copy(buf, out_hbm.at[c * _NW + wid], sem)

    for b in range(3):
        build(b, bufs[b])
        write(b, bufs[b], sems[b]).start()

    def step(i, carry):
        for b in range(3):
            c = 3 + 3 * i + b
            write(c - 3, bufs[b], sems[b]).wait()
            build(c, bufs[b])
            write(c, bufs[b], sems[b]).start()
        return carry

    lax.fori_loop(0, (_B_PER_W - 3) // 3, step, 0)
    for b in range(2):
        c = _B_PER_W - 2 + b
        write(c - 3, bufs[b], sems[b]).wait()
        build(c, bufs[b])
        write(c, bufs[b], sems[b]).start()

    for b in range(3):
        write(0, bufs[b], sems[b]).wait()


@jax.jit
def _gate_lookup(inputs, gate_weight):
    mesh = plsc.VectorSubcoreMesh(core_axis_name="c", subcore_axis_name="s")
    run = pl.kernel(
        _gate_body,
        out_type=jax.ShapeDtypeStruct((_BATCH, _ATOMS, _DOUT), jnp.float32),
        mesh=mesh,
        scratch_types=[
            pltpu.VMEM((_B_PER_W * _ATOMS + _LANES,), jnp.int32),
            pltpu.VMEM((_NROWS, _DOUT), jnp.float32),
            pltpu.VMEM((_ATOMS, _DOUT), jnp.float32),
            pltpu.VMEM((_ATOMS, _DOUT), jnp.float32),
            pltpu.VMEM((_ATOMS, _DOUT), jnp.float32),
            pltpu.SemaphoreType.DMA,
            pltpu.SemaphoreType.DMA,
            pltpu.SemaphoreType.DMA,
        ],
    )
    idx4 = inputs.reshape(_B_PER_W, _NW, _ATOMS).transpose(1, 0, 2)
    return run(idx4.reshape(-1), gate_weight)


def kernel(inputs, gate_weight):
    return _gate_lookup(inputs, gate_weight)


# speedup vs baseline: 2.2114x; 1.0013x over previous
import jax
import jax.numpy as jnp
from jax import lax
from jax.experimental import pallas as pl
from jax.experimental.pallas import tpu as pltpu
from jax.experimental.pallas import tpu_sc as plsc

_BATCH = 4096
_ATOMS = 50
_DOUT = 640
_NROWS = 10

_NC = 2
_NS = 16
_NW = _NC * _NS

_B_PER_W = _BATCH // _NW
_LANES = 16
_VPR = _DOUT // _LANES


def _gate_body(idx_hbm, table_hbm, out_hbm, idx_v, table_v, buf0, buf1, buf2,
               wsem0, wsem1, wsem2):
    wid = lax.axis_index("s") * _NC + lax.axis_index("c")

    pltpu.sync_copy(table_hbm, table_v)
    pltpu.sync_copy(idx_hbm.at[pl.ds(wid * _B_PER_W * _ATOMS,
                                     _B_PER_W * _ATOMS)],
                    idx_v.at[pl.ds(0, _B_PER_W * _ATOMS)])

    bufs = (buf0, buf1, buf2)
    sems = (wsem0, wsem1, wsem2)

    def build(c, buf):
        G = 8
        R = 5

        def rows(g, carry):
            r0 = g * R
            ivs = [idx_v[pl.ds(c * _ATOMS + r0 + k, _LANES)][0]
                   for k in range(R)]
            slots = [(k, j) for k in range(R) for j in range(_VPR)]

            def load(p):
                k, j = slots[p]
                return table_v[ivs[k], pl.ds(j * _LANES, _LANES)]

            vals = [load(p) for p in range(G)]
            for p in range(R * _VPR):
                k, j = slots[p]
                buf[r0 + k, pl.ds(j * _LANES, _LANES)] = vals[p % G]
                if p + G < R * _VPR:
                    vals[p % G] = load(p + G)
            return carry
        lax.fori_loop(0, _ATOMS // R, rows, 0)

    def write(c, buf, sem):
        return pltpu.make_async_copy(buf, out_hbm.at[c * _NW + wid], sem)

    for b in range(3):
        build(b, bufs[b])
        write(b, bufs[b], sems[b]).start()

    def step(i, carry):
        for b in range(3):
            c = 3 + 3 * i + b
            write(c - 3, bufs[b], sems[b]).wait()
            build(c, bufs[b])
            write(c, bufs[b], sems[b]).start()
        return carry

    lax.fori_loop(0, (_B_PER_W - 3) // 3, step, 0)
    for b in range(2):
        c = _B_PER_W - 2 + b
        write(c - 3, bufs[b], sems[b]).wait()
        build(c, bufs[b])
        write(c, bufs[b], sems[b]).start()

    for b in range(3):
        write(0, bufs[b], sems[b]).wait()


@jax.jit
def _gate_lookup(inputs, gate_weight):
    mesh = plsc.VectorSubcoreMesh(core_axis_name="c", subcore_axis_name="s")
    run = pl.kernel(
        _gate_body,
        out_type=jax.ShapeDtypeStruct((_BATCH, _ATOMS, _DOUT), jnp.float32),
        mesh=mesh,
        scratch_types=[
            pltpu.VMEM((_B_PER_W * _ATOMS + _LANES,), jnp.int32),
            pltpu.VMEM((_NROWS, _DOUT), jnp.float32),
            pltpu.VMEM((_ATOMS, _DOUT), jnp.float32),
            pltpu.VMEM((_ATOMS, _DOUT), jnp.float32),
            pltpu.VMEM((_ATOMS, _DOUT), jnp.float32),
            pltpu.SemaphoreType.DMA,
            pltpu.SemaphoreType.DMA,
            pltpu.SemaphoreType.DMA,
        ],
    )
    idx4 = inputs.reshape(_B_PER_W, _NW, _ATOMS).transpose(1, 0, 2)
    return run(idx4.reshape(-1), gate_weight)


def kernel(inputs, gate_weight):
    return _gate_lookup(inputs, gate_weight)
